# Initial kernel scaffold; baseline (speedup 1.0000x reference)
import jax
import jax.numpy as jnp
from jax.experimental import pallas as pl


def kernel(x, Wg, bg, W1, b1, W2, b2):
    raise NotImplementedError("write your pallas kernel here")


# speedup vs baseline: 1.0608x; 1.0608x over previous
import jax
import jax.numpy as jnp
from jax.experimental import pallas as pl
from jax.experimental.pallas import tpu as pltpu

_N = 128
_D = 768
_H = 768
_E = 64
_K = 2


def _moe_body(x_ref, wg_ref, bg_ref, w1_ref, b1_ref, w2_ref, b2_ref,
              out_ref, loss_ref, comb_ref):
    e = pl.program_id(0)

    @pl.when(e == 0)
    def _routing():
        logits = jnp.dot(x_ref[...], wg_ref[...],
                         preferred_element_type=jnp.float32) + bg_ref[...]
        m = jnp.max(logits, axis=-1, keepdims=True)
        p = jnp.exp(logits - m)
        gates = p / jnp.sum(p, axis=-1, keepdims=True)
        col = jax.lax.broadcasted_iota(jnp.int32, (_N, _E), 1)
        i1 = jnp.argmax(gates, axis=-1)
        oh1 = col == i1[:, None]
        g2 = jnp.where(oh1, -jnp.inf, gates)
        i2 = jnp.argmax(g2, axis=-1)
        oh2 = col == i2[:, None]
        sel = oh1 | oh2
        comb_ref[...] = jnp.where(sel, gates, 0.0)
        load = jnp.sum(sel.astype(jnp.float32), axis=0)
        frac = load / jnp.float32(_N * _K)
        loss_ref[...] = jnp.full((1, 1), jnp.sum(frac * frac), jnp.float32)
        out_ref[...] = jnp.zeros_like(out_ref)

    col = jax.lax.broadcasted_iota(jnp.int32, (_N, _E), 1)
    w = jnp.sum(jnp.where(col == e, comb_ref[...], 0.0), axis=1)
    h = jnp.maximum(
        jnp.dot(x_ref[...], w1_ref[0], preferred_element_type=jnp.float32)
        + b1_ref[0], 0.0)
    y = jnp.dot(h, w2_ref[0], preferred_element_type=jnp.float32) + b2_ref[0]
    out_ref[...] += w[:, None] * y


def kernel(x, Wg, bg, W1, b1, W2, b2):
    out, loss = pl.pallas_call(
        _moe_body,
        grid=(_E,),
        in_specs=[
            pl.BlockSpec((_N, _D), lambda e: (0, 0)),
            pl.BlockSpec((_D, _E), lambda e: (0, 0)),
            pl.BlockSpec((1, _E), lambda e: (0, 0)),
            pl.BlockSpec((1, _D, _H), lambda e: (e, 0, 0)),
            pl.BlockSpec((1, 1, _H), lambda e: (e, 0, 0)),
            pl.BlockSpec((1, _H, _D), lambda e: (e, 0, 0)),
            pl.BlockSpec((1, 1, _D), lambda e: (e, 0, 0)),
        ],
        out_specs=[
            pl.BlockSpec((_N, _D), lambda e: (0, 0)),
            pl.BlockSpec((1, 1), lambda e: (0, 0)),
        ],
        out_shape=[
            jax.ShapeDtypeStruct((_N, _D), jnp.float32),
            jax.ShapeDtypeStruct((1, 1), jnp.float32),
        ],
        scratch_shapes=[pltpu.VMEM((_N, _E), jnp.float32)],
        compiler_params=pltpu.CompilerParams(
            dimension_semantics=("arbitrary",),
        ),
    )(x, Wg, bg.reshape(1, _E), W1, b1.reshape(_E, 1, _H), W2,
      b2.reshape(_E, 1, _D))
    return out, loss.reshape(())


# speedup vs baseline: 1.0630x; 1.0021x over previous
import jax
import jax.numpy as jnp
from jax.experimental import pallas as pl
from jax.experimental.pallas import tpu as pltpu

_N = 128
_D = 768
_H = 768
_E = 64
_K = 2


def _moe_body(x_ref, wg_ref, bg_ref, w1_ref, b1_ref, w2_ref, b2_ref,
              out_ref, loss_ref, comb_ref):
    e = pl.program_id(0)

    @pl.when(e == 0)
    def _routing():
        logits = jnp.dot(x_ref[...], wg_ref[...],
                         preferred_element_type=jnp.float32) + bg_ref[...]
        m = jnp.max(logits, axis=-1, keepdims=True)
        p = jnp.exp(logits - m)
        gates = p / jnp.sum(p, axis=-1, keepdims=True)
        col = jax.lax.broadcasted_iota(jnp.int32, (_N, _E), 1)
        i1 = jnp.argmax(gates, axis=-1)
        oh1 = col == i1[:, None]
        g2 = jnp.where(oh1, -jnp.inf, gates)
        i2 = jnp.argmax(g2, axis=-1)
        oh2 = col == i2[:, None]
        sel = oh1 | oh2
        comb_ref[...] = jnp.where(sel, gates, 0.0)
        load = jnp.sum(sel.astype(jnp.float32), axis=0)
        frac = load / jnp.float32(_N * _K)
        loss_ref[...] = jnp.full((1, 1), jnp.sum(frac * frac), jnp.float32)
        out_ref[...] = jnp.zeros_like(out_ref)

    col = jax.lax.broadcasted_iota(jnp.int32, (_N, _E), 1)
    w = jnp.sum(jnp.where(col == e, comb_ref[...], 0.0), axis=1)
    xb = x_ref[...].astype(jnp.bfloat16)
    h = jnp.maximum(
        jnp.dot(xb, w1_ref[0].astype(jnp.bfloat16),
                preferred_element_type=jnp.float32) + b1_ref[0], 0.0)
    y = jnp.dot(h.astype(jnp.bfloat16), w2_ref[0].astype(jnp.bfloat16),
                preferred_element_type=jnp.float32) + b2_ref[0]
    out_ref[...] += w[:, None] * y


def kernel(x, Wg, bg, W1, b1, W2, b2):
    out, loss = pl.pallas_call(
        _moe_body,
        grid=(_E,),
        in_specs=[
            pl.BlockSpec((_N, _D), lambda e: (0, 0)),
            pl.BlockSpec((_D, _E), lambda e: (0, 0)),
            pl.BlockSpec((1, _E), lambda e: (0, 0)),
            pl.BlockSpec((1, _D, _H), lambda e: (e, 0, 0)),
            pl.BlockSpec((1, 1, _H), lambda e: (e, 0, 0)),
            pl.BlockSpec((1, _H, _D), lambda e: (e, 0, 0)),
            pl.BlockSpec((1, 1, _D), lambda e: (e, 0, 0)),
        ],
        out_specs=[
            pl.BlockSpec((_N, _D), lambda e: (0, 0)),
            pl.BlockSpec((1, 1), lambda e: (0, 0)),
        ],
        out_shape=[
            jax.ShapeDtypeStruct((_N, _D), jnp.float32),
            jax.ShapeDtypeStruct((1, 1), jnp.float32),
        ],
        scratch_shapes=[pltpu.VMEM((_N, _E), jnp.float32)],
        compiler_params=pltpu.CompilerParams(
            dimension_semantics=("arbitrary",),
        ),
    )(x, Wg, bg.reshape(1, _E), W1, b1.reshape(_E, 1, _H), W2,
      b2.reshape(_E, 1, _D))
    return out, loss.reshape(())


# speedup vs baseline: 1.1955x; 1.1247x over previous
import jax
import jax.numpy as jnp
from jax.experimental import pallas as pl
from jax.experimental.pallas import tpu as pltpu

_N = 128
_D = 768
_H = 768
_E = 64
_K = 2
_EPB = 2


def _moe_body(x_ref, wg_ref, bg_ref, w1_ref, b1_ref, w2_ref, b2_ref,
              out_ref, loss_ref, comb_ref):
    e = pl.program_id(0)

    @pl.when(e == 0)
    def _routing():
        logits = jnp.dot(x_ref[...], wg_ref[...],
                         preferred_element_type=jnp.float32) + bg_ref[...]
        m = jnp.max(logits, axis=-1, keepdims=True)
        p = jnp.exp(logits - m)
        gates = p / jnp.sum(p, axis=-1, keepdims=True)
        col = jax.lax.broadcasted_iota(jnp.int32, (_N, _E), 1)
        i1 = jnp.argmax(gates, axis=-1)
        oh1 = col == i1[:, None]
        g2 = jnp.where(oh1, -jnp.inf, gates)
        i2 = jnp.argmax(g2, axis=-1)
        oh2 = col == i2[:, None]
        sel = oh1 | oh2
        comb_ref[...] = jnp.where(sel, gates, 0.0)
        load = jnp.sum(sel.astype(jnp.float32), axis=0)
        frac = load / jnp.float32(_N * _K)
        loss_ref[...] = jnp.full((1, 1), jnp.sum(frac * frac), jnp.float32)
        out_ref[...] = jnp.zeros_like(out_ref)

    col = jax.lax.broadcasted_iota(jnp.int32, (_N, _E), 1)
    xb = x_ref[...].astype(jnp.bfloat16)
    acc = jnp.zeros((_N, _D), jnp.float32)
    for j in range(_EPB):
        w = jnp.sum(jnp.where(col == e * _EPB + j, comb_ref[...], 0.0),
                    axis=1)
        h = jnp.maximum(
            jnp.dot(xb, w1_ref[j].astype(jnp.bfloat16),
                    preferred_element_type=jnp.float32) + b1_ref[j], 0.0)
        y = jnp.dot(h.astype(jnp.bfloat16), w2_ref[j].astype(jnp.bfloat16),
                    preferred_element_type=jnp.float32) + b2_ref[j]
        acc += w[:, None] * y
    out_ref[...] += acc


def kernel(x, Wg, bg, W1, b1, W2, b2):
    out, loss = pl.pallas_call(
        _moe_body,
        grid=(_E // _EPB,),
        in_specs=[
            pl.BlockSpec((_N, _D), lambda e: (0, 0)),
            pl.BlockSpec((_D, _E), lambda e: (0, 0)),
            pl.BlockSpec((1, _E), lambda e: (0, 0)),
            pl.BlockSpec((_EPB, _D, _H), lambda e: (e, 0, 0)),
            pl.BlockSpec((_EPB, 1, _H), lambda e: (e, 0, 0)),
            pl.BlockSpec((_EPB, _H, _D), lambda e: (e, 0, 0)),
            pl.BlockSpec((_EPB, 1, _D), lambda e: (e, 0, 0)),
        ],
        out_specs=[
            pl.BlockSpec((_N, _D), lambda e: (0, 0)),
            pl.BlockSpec((1, 1), lambda e: (0, 0)),
        ],
        out_shape=[
            jax.ShapeDtypeStruct((_N, _D), jnp.float32),
            jax.ShapeDtypeStruct((1, 1), jnp.float32),
        ],
        scratch_shapes=[pltpu.VMEM((_N, _E), jnp.float32)],
        compiler_params=pltpu.CompilerParams(
            dimension_semantics=("arbitrary",),
        ),
    )(x, Wg, bg.reshape(1, _E), W1, b1.reshape(_E, 1, _H), W2,
      b2.reshape(_E, 1, _D))
    return out, loss.reshape(())
